# initial kernel scaffold (unmeasured)
import jax
import jax.numpy as jnp
from jax import lax
from jax.experimental import pallas as pl
from jax.experimental.pallas import tpu as pltpu


def kernel(
    x,
):
    def body(*refs):
        pass

    out_shape = jax.ShapeDtypeStruct(..., jnp.float32)
    return pl.pallas_call(body, out_shape=out_shape)(...)


# baseline (device time: 330263 ns/iter reference)
import jax
import jax.numpy as jnp
from jax import lax
from jax.experimental import pallas as pl
from jax.experimental.pallas import tpu as pltpu

N_DEV = 4

_DeviceIdType = getattr(pl, "DeviceIdType", None) or pltpu.DeviceIdType
_sem_signal = getattr(pl, "semaphore_signal", None) or pltpu.semaphore_signal
_sem_wait = getattr(pl, "semaphore_wait", None) or pltpu.semaphore_wait
_CompilerParams = getattr(pltpu, "CompilerParams", None) or getattr(
    pltpu, "TPUCompilerParams"
)
_ANY = getattr(pltpu, "ANY", None) or pl.ANY


def kernel(x):
    _, m, n = x.shape
    ch = n // N_DEV

    xb = x.reshape(m, n).astype(jnp.bfloat16)

    def body(x_hbm, out_ref, chunk_ref, comm_ref, local_sem, send_sems, recv_sems):
        p = lax.axis_index("i")
        left = lax.rem(p + N_DEV - 1, N_DEV)
        right = lax.rem(p + 1, N_DEV)

        barrier_sem = pltpu.get_barrier_semaphore()
        for nbr in (left, right):
            _sem_signal(
                barrier_sem,
                inc=1,
                device_id=(nbr,),
                device_id_type=_DeviceIdType.MESH,
            )
        _sem_wait(barrier_sem, 2)

        def load_chunk(c):
            cp = pltpu.make_async_copy(
                x_hbm.at[:, pl.ds(c * ch, ch)], chunk_ref, local_sem
            )
            cp.start()
            cp.wait()

        load_chunk(lax.rem(p + 3, N_DEV))

        for h in range(N_DEV - 1):
            src = chunk_ref if h == 0 else comm_ref.at[h - 1]
            rdma = pltpu.make_async_remote_copy(
                src_ref=src,
                dst_ref=comm_ref.at[h],
                send_sem=send_sems.at[h],
                recv_sem=recv_sems.at[h],
                device_id=(right,),
                device_id_type=_DeviceIdType.MESH,
            )
            rdma.start()
            rdma.wait()

            c_recv = lax.rem(p + 2 - h, N_DEV)
            load_chunk(c_recv)
            if h < N_DEV - 2:
                comm_ref[h, :, :] = comm_ref[h, :, :] + chunk_ref[:, :]
            else:
                out_ref[:, :] = comm_ref[h, :, :] + chunk_ref[:, :]

    return pl.pallas_call(
        body,
        out_shape=jax.ShapeDtypeStruct((m, ch), jnp.bfloat16),
        in_specs=[pl.BlockSpec(memory_space=_ANY)],
        out_specs=pl.BlockSpec(memory_space=pltpu.VMEM),
        scratch_shapes=[
            pltpu.VMEM((m, ch), jnp.bfloat16),
            pltpu.VMEM((N_DEV - 1, m, ch), jnp.bfloat16),
            pltpu.SemaphoreType.DMA,
            pltpu.SemaphoreType.DMA((N_DEV - 1,)),
            pltpu.SemaphoreType.DMA((N_DEV - 1,)),
        ],
        compiler_params=_CompilerParams(collective_id=0),
    )(xb)


# device time: 182933 ns/iter; 1.8054x vs baseline; 1.8054x over previous
import jax
import jax.numpy as jnp
from jax import lax
from jax.experimental import pallas as pl
from jax.experimental.pallas import tpu as pltpu

N_DEV = 4

_DeviceIdType = getattr(pl, "DeviceIdType", None) or pltpu.DeviceIdType
_sem_signal = getattr(pl, "semaphore_signal", None) or pltpu.semaphore_signal
_sem_wait = getattr(pl, "semaphore_wait", None) or pltpu.semaphore_wait
_CompilerParams = getattr(pltpu, "CompilerParams", None) or getattr(
    pltpu, "TPUCompilerParams"
)
_ANY = getattr(pltpu, "ANY", None) or pl.ANY


def kernel(x):
    _, m, n = x.shape
    ch = n // N_DEV
    mh = m // 2

    xb = x.reshape(m, n).astype(jnp.bfloat16)

    def body(
        x_hbm,
        out_ref,
        stage_ref,
        comm_cw,
        comm_ccw,
        local_sems,
        send_sems,
        recv_sems,
    ):
        p = lax.axis_index("i")
        left = lax.rem(p + N_DEV - 1, N_DEV)
        right = lax.rem(p + 1, N_DEV)

        barrier_sem = pltpu.get_barrier_semaphore()
        for nbr in (left, right):
            _sem_signal(
                barrier_sem,
                inc=1,
                device_id=(nbr,),
                device_id_type=_DeviceIdType.MESH,
            )
        _sem_wait(barrier_sem, 2)

        def col(c):
            return pl.ds(c * ch, ch)

        for h in range(N_DEV - 1):
            if h == 0:
                src_cw = x_hbm.at[pl.ds(0, mh), col(lax.rem(p + 3, N_DEV))]
                src_ccw = x_hbm.at[pl.ds(mh, mh), col(lax.rem(p + 1, N_DEV))]
            else:
                src_cw = comm_cw.at[h - 1]
                src_ccw = comm_ccw.at[h - 1]
            rdma_cw = pltpu.make_async_remote_copy(
                src_ref=src_cw,
                dst_ref=comm_cw.at[h],
                send_sem=send_sems.at[0, h],
                recv_sem=recv_sems.at[0, h],
                device_id=(right,),
                device_id_type=_DeviceIdType.MESH,
            )
            rdma_ccw = pltpu.make_async_remote_copy(
                src_ref=src_ccw,
                dst_ref=comm_ccw.at[h],
                send_sem=send_sems.at[1, h],
                recv_sem=recv_sems.at[1, h],
                device_id=(left,),
                device_id_type=_DeviceIdType.MESH,
            )
            rdma_cw.start()
            rdma_ccw.start()

            c_cw = lax.rem(p + 2 - h + N_DEV, N_DEV)
            c_ccw = lax.rem(p + 2 + h, N_DEV)
            ld_cw = pltpu.make_async_copy(
                x_hbm.at[pl.ds(0, mh), col(c_cw)], stage_ref.at[0], local_sems.at[0]
            )
            ld_ccw = pltpu.make_async_copy(
                x_hbm.at[pl.ds(mh, mh), col(c_ccw)], stage_ref.at[1], local_sems.at[1]
            )
            ld_cw.start()
            ld_ccw.start()

            rdma_cw.wait()
            rdma_ccw.wait()
            ld_cw.wait()
            ld_ccw.wait()

            if h < N_DEV - 2:
                comm_cw[h, :, :] = comm_cw[h, :, :] + stage_ref[0, :, :]
                comm_ccw[h, :, :] = comm_ccw[h, :, :] + stage_ref[1, :, :]
            else:
                out_ref[0:mh, :] = comm_cw[h, :, :] + stage_ref[0, :, :]
                out_ref[mh : 2 * mh, :] = comm_ccw[h, :, :] + stage_ref[1, :, :]

    return pl.pallas_call(
        body,
        out_shape=jax.ShapeDtypeStruct((m, ch), jnp.bfloat16),
        in_specs=[pl.BlockSpec(memory_space=_ANY)],
        out_specs=pl.BlockSpec(memory_space=pltpu.VMEM),
        scratch_shapes=[
            pltpu.VMEM((2, mh, ch), jnp.bfloat16),
            pltpu.VMEM((N_DEV - 1, mh, ch), jnp.bfloat16),
            pltpu.VMEM((N_DEV - 1, mh, ch), jnp.bfloat16),
            pltpu.SemaphoreType.DMA((2,)),
            pltpu.SemaphoreType.DMA((2, N_DEV - 1)),
            pltpu.SemaphoreType.DMA((2, N_DEV - 1)),
        ],
        compiler_params=_CompilerParams(collective_id=0),
    )(xb)


# device time: 163246 ns/iter; 2.0231x vs baseline; 1.1206x over previous
import jax
import jax.numpy as jnp
from jax import lax
from jax.experimental import pallas as pl
from jax.experimental.pallas import tpu as pltpu

N_DEV = 4

_DeviceIdType = getattr(pl, "DeviceIdType", None) or pltpu.DeviceIdType
_sem_signal = getattr(pl, "semaphore_signal", None) or pltpu.semaphore_signal
_sem_wait = getattr(pl, "semaphore_wait", None) or pltpu.semaphore_wait
_CompilerParams = getattr(pltpu, "CompilerParams", None) or getattr(
    pltpu, "TPUCompilerParams"
)
_ANY = getattr(pltpu, "ANY", None) or pl.ANY


def kernel(x):
    _, m, n = x.shape
    ch = n // N_DEV
    mh = m // 2

    def body(
        x_hbm,
        out_ref,
        stage_ref,
        send0_ref,
        comm_cw,
        comm_ccw,
        local_sems,
        send_sems,
        recv_sems,
    ):
        p = lax.axis_index("i")
        left = lax.rem(p + N_DEV - 1, N_DEV)
        right = lax.rem(p + 1, N_DEV)

        barrier_sem = pltpu.get_barrier_semaphore()
        for nbr in (left, right):
            _sem_signal(
                barrier_sem,
                inc=1,
                device_id=(nbr,),
                device_id_type=_DeviceIdType.MESH,
            )
        _sem_wait(barrier_sem, 2)

        def col(c):
            return pl.ds(c * ch, ch)

        def load(dir_, c):
            row0 = 0 if dir_ == 0 else mh
            return pltpu.make_async_copy(
                x_hbm.at[0, pl.ds(row0, mh), col(c)],
                stage_ref.at[dir_],
                local_sems.at[dir_],
            )

        ld0 = load(0, lax.rem(p + 3, N_DEV))
        ld1 = load(1, lax.rem(p + 1, N_DEV))
        ld0.start()
        ld1.start()
        ld0.wait()
        send0_ref[0, :, :] = stage_ref[0, :, :].astype(jnp.bfloat16)
        ld1.wait()
        send0_ref[1, :, :] = stage_ref[1, :, :].astype(jnp.bfloat16)

        for h in range(N_DEV - 1):
            rdma_cw = pltpu.make_async_remote_copy(
                src_ref=send0_ref.at[0] if h == 0 else comm_cw.at[h - 1],
                dst_ref=comm_cw.at[h],
                send_sem=send_sems.at[0, h],
                recv_sem=recv_sems.at[0, h],
                device_id=(right,),
                device_id_type=_DeviceIdType.MESH,
            )
            rdma_ccw = pltpu.make_async_remote_copy(
                src_ref=send0_ref.at[1] if h == 0 else comm_ccw.at[h - 1],
                dst_ref=comm_ccw.at[h],
                send_sem=send_sems.at[1, h],
                recv_sem=recv_sems.at[1, h],
                device_id=(left,),
                device_id_type=_DeviceIdType.MESH,
            )
            rdma_cw.start()
            rdma_ccw.start()

            ld_cw = load(0, lax.rem(p + 2 - h + N_DEV, N_DEV))
            ld_ccw = load(1, lax.rem(p + 2 + h, N_DEV))
            ld_cw.start()
            ld_ccw.start()

            rdma_cw.wait()
            rdma_ccw.wait()
            ld_cw.wait()
            ld_ccw.wait()

            if h < N_DEV - 2:
                comm_cw[h, :, :] = comm_cw[h, :, :] + stage_ref[0, :, :].astype(
                    jnp.bfloat16
                )
                comm_ccw[h, :, :] = comm_ccw[h, :, :] + stage_ref[1, :, :].astype(
                    jnp.bfloat16
                )
            else:
                out_ref[0:mh, :] = comm_cw[h, :, :] + stage_ref[0, :, :].astype(
                    jnp.bfloat16
                )
                out_ref[mh : 2 * mh, :] = comm_ccw[h, :, :] + stage_ref[
                    1, :, :
                ].astype(jnp.bfloat16)

    return pl.pallas_call(
        body,
        out_shape=jax.ShapeDtypeStruct((m, ch), jnp.bfloat16),
        in_specs=[pl.BlockSpec(memory_space=_ANY)],
        out_specs=pl.BlockSpec(memory_space=pltpu.VMEM),
        scratch_shapes=[
            pltpu.VMEM((2, mh, ch), jnp.float32),
            pltpu.VMEM((2, mh, ch), jnp.bfloat16),
            pltpu.VMEM((N_DEV - 1, mh, ch), jnp.bfloat16),
            pltpu.VMEM((N_DEV - 1, mh, ch), jnp.bfloat16),
            pltpu.SemaphoreType.DMA((2,)),
            pltpu.SemaphoreType.DMA((2, N_DEV - 1)),
            pltpu.SemaphoreType.DMA((2, N_DEV - 1)),
        ],
        compiler_params=_CompilerParams(
            collective_id=0, vmem_limit_bytes=60 * 1024 * 1024
        ),
    )(x)


# device time: 155598 ns/iter; 2.1225x vs baseline; 1.0492x over previous
import jax
import jax.numpy as jnp
from jax import lax
from jax.experimental import pallas as pl
from jax.experimental.pallas import tpu as pltpu

N_DEV = 4
SUB = 2

_DeviceIdType = getattr(pl, "DeviceIdType", None) or pltpu.DeviceIdType
_sem_signal = getattr(pl, "semaphore_signal", None) or pltpu.semaphore_signal
_sem_wait = getattr(pl, "semaphore_wait", None) or pltpu.semaphore_wait
_CompilerParams = getattr(pltpu, "CompilerParams", None) or getattr(
    pltpu, "TPUCompilerParams"
)
_ANY = getattr(pltpu, "ANY", None) or pl.ANY


def kernel(x):
    _, m, n = x.shape
    ch = n // N_DEV
    mh = m // 2
    qm = mh // SUB

    def body(
        x_hbm,
        out_ref,
        stage_ref,
        send0_ref,
        comm_cw,
        comm_ccw,
        local_sems,
        send_sems,
        recv_sems,
    ):
        p = lax.axis_index("i")
        left = lax.rem(p + N_DEV - 1, N_DEV)
        right = lax.rem(p + 1, N_DEV)
        comm = (comm_cw, comm_ccw)
        peer = (right, left)

        barrier_sem = pltpu.get_barrier_semaphore()
        for nbr in (left, right):
            _sem_signal(
                barrier_sem,
                inc=1,
                device_id=(nbr,),
                device_id_type=_DeviceIdType.MESH,
            )
        _sem_wait(barrier_sem, 2)

        def col(c):
            return pl.ds(c * ch, ch)

        def rows(s):
            return pl.ds(s * qm, qm)

        def send_chunk(d, h):
            return lax.rem(p + 3 - h + N_DEV, N_DEV) if d == 0 else lax.rem(
                p + 1 + h, N_DEV
            )

        def recv_chunk(d, h):
            return lax.rem(p + 2 - h + N_DEV, N_DEV) if d == 0 else lax.rem(
                p + 2 + h, N_DEV
            )

        def load(d, s, c):
            return pltpu.make_async_copy(
                x_hbm.at[0, pl.ds(d * mh + s * qm, qm), col(c)],
                stage_ref.at[d, rows(s)],
                local_sems.at[d, s],
            )

        def make_rdma(d, h, s):
            src = send0_ref.at[d, rows(s)] if h == 0 else comm[d].at[h - 1, rows(s)]
            return pltpu.make_async_remote_copy(
                src_ref=src,
                dst_ref=comm[d].at[h, rows(s)],
                send_sem=send_sems.at[d, h, s],
                recv_sem=recv_sems.at[d, h, s],
                device_id=(peer[d],),
                device_id_type=_DeviceIdType.MESH,
            )

        rdmas = {}
        loads = {}

        for d in (0, 1):
            for s in range(SUB):
                loads[(d, -1, s)] = load(d, s, send_chunk(d, 0))
                loads[(d, -1, s)].start()
        for s in range(SUB):
            for d in (0, 1):
                loads[(d, -1, s)].wait()
                send0_ref[d, rows(s)] = stage_ref[d, rows(s)].astype(jnp.bfloat16)
                rdmas[(d, 0, s)] = make_rdma(d, 0, s)
                rdmas[(d, 0, s)].start()
                loads[(d, 0, s)] = load(d, s, recv_chunk(d, 0))
                loads[(d, 0, s)].start()

        for h in range(N_DEV - 1):
            for s in range(SUB):
                for d in (0, 1):
                    rdmas[(d, h, s)].wait_recv()
                    loads[(d, h, s)].wait()
                    acc = comm[d][h, rows(s)] + stage_ref[d, rows(s)].astype(
                        jnp.bfloat16
                    )
                    if h < N_DEV - 2:
                        comm[d][h, rows(s)] = acc
                        rdmas[(d, h + 1, s)] = make_rdma(d, h + 1, s)
                        rdmas[(d, h + 1, s)].start()
                        loads[(d, h + 1, s)] = load(d, s, recv_chunk(d, h + 1))
                        loads[(d, h + 1, s)].start()
                    else:
                        out_ref[pl.ds(d * mh + s * qm, qm), :] = acc

        for key in rdmas:
            rdmas[key].wait_send()

    return pl.pallas_call(
        body,
        out_shape=jax.ShapeDtypeStruct((m, ch), jnp.bfloat16),
        in_specs=[pl.BlockSpec(memory_space=_ANY)],
        out_specs=pl.BlockSpec(memory_space=pltpu.VMEM),
        scratch_shapes=[
            pltpu.VMEM((2, mh, ch), jnp.float32),
            pltpu.VMEM((2, mh, ch), jnp.bfloat16),
            pltpu.VMEM((N_DEV - 1, mh, ch), jnp.bfloat16),
            pltpu.VMEM((N_DEV - 1, mh, ch), jnp.bfloat16),
            pltpu.SemaphoreType.DMA((2, SUB)),
            pltpu.SemaphoreType.DMA((2, N_DEV - 1, SUB)),
            pltpu.SemaphoreType.DMA((2, N_DEV - 1, SUB)),
        ],
        compiler_params=_CompilerParams(
            collective_id=0, vmem_limit_bytes=60 * 1024 * 1024
        ),
    )(x)


# device time: 155004 ns/iter; 2.1307x vs baseline; 1.0038x over previous
import jax
import jax.numpy as jnp
from jax import lax
from jax.experimental import pallas as pl
from jax.experimental.pallas import tpu as pltpu

N_DEV = 4
SUB = 4

_DeviceIdType = getattr(pl, "DeviceIdType", None) or pltpu.DeviceIdType
_sem_signal = getattr(pl, "semaphore_signal", None) or pltpu.semaphore_signal
_sem_wait = getattr(pl, "semaphore_wait", None) or pltpu.semaphore_wait
_CompilerParams = getattr(pltpu, "CompilerParams", None) or getattr(
    pltpu, "TPUCompilerParams"
)
_ANY = getattr(pltpu, "ANY", None) or pl.ANY


def kernel(x):
    _, m, n = x.shape
    ch = n // N_DEV
    mh = m // 2
    qm = mh // SUB

    def body(
        x_hbm,
        out_ref,
        stage_ref,
        send0_ref,
        comm_cw,
        comm_ccw,
        local_sems,
        send_sems,
        recv_sems,
    ):
        p = lax.axis_index("i")
        left = lax.rem(p + N_DEV - 1, N_DEV)
        right = lax.rem(p + 1, N_DEV)
        comm = (comm_cw, comm_ccw)
        peer = (right, left)

        barrier_sem = pltpu.get_barrier_semaphore()
        for nbr in (left, right):
            _sem_signal(
                barrier_sem,
                inc=1,
                device_id=(nbr,),
                device_id_type=_DeviceIdType.MESH,
            )
        _sem_wait(barrier_sem, 2)

        def col(c):
            return pl.ds(c * ch, ch)

        def rows(s):
            return pl.ds(s * qm, qm)

        def send_chunk(d, h):
            return lax.rem(p + 3 - h + N_DEV, N_DEV) if d == 0 else lax.rem(
                p + 1 + h, N_DEV
            )

        def recv_chunk(d, h):
            return lax.rem(p + 2 - h + N_DEV, N_DEV) if d == 0 else lax.rem(
                p + 2 + h, N_DEV
            )

        def load(d, s, c):
            return pltpu.make_async_copy(
                x_hbm.at[0, pl.ds(d * mh + s * qm, qm), col(c)],
                stage_ref.at[d, rows(s)],
                local_sems.at[d, s],
            )

        def make_rdma(d, h, s):
            src = send0_ref.at[d, rows(s)] if h == 0 else comm[d].at[h - 1, rows(s)]
            return pltpu.make_async_remote_copy(
                src_ref=src,
                dst_ref=comm[d].at[h, rows(s)],
                send_sem=send_sems.at[d, h, s],
                recv_sem=recv_sems.at[d, h, s],
                device_id=(peer[d],),
                device_id_type=_DeviceIdType.MESH,
            )

        rdmas = {}
        loads = {}

        for d in (0, 1):
            for s in range(SUB):
                loads[(d, -1, s)] = load(d, s, send_chunk(d, 0))
                loads[(d, -1, s)].start()
        for s in range(SUB):
            for d in (0, 1):
                loads[(d, -1, s)].wait()
                send0_ref[d, rows(s)] = stage_ref[d, rows(s)].astype(jnp.bfloat16)
                rdmas[(d, 0, s)] = make_rdma(d, 0, s)
                rdmas[(d, 0, s)].start()
                loads[(d, 0, s)] = load(d, s, recv_chunk(d, 0))
                loads[(d, 0, s)].start()

        for h in range(N_DEV - 1):
            for s in range(SUB):
                for d in (0, 1):
                    rdmas[(d, h, s)].wait_recv()
                    loads[(d, h, s)].wait()
                    acc = comm[d][h, rows(s)] + stage_ref[d, rows(s)].astype(
                        jnp.bfloat16
                    )
                    if h < N_DEV - 2:
                        comm[d][h, rows(s)] = acc
                        rdmas[(d, h + 1, s)] = make_rdma(d, h + 1, s)
                        rdmas[(d, h + 1, s)].start()
                        loads[(d, h + 1, s)] = load(d, s, recv_chunk(d, h + 1))
                        loads[(d, h + 1, s)].start()
                    else:
                        out_ref[pl.ds(d * mh + s * qm, qm), :] = acc

        for key in rdmas:
            rdmas[key].wait_send()

    return pl.pallas_call(
        body,
        out_shape=jax.ShapeDtypeStruct((m, ch), jnp.bfloat16),
        in_specs=[pl.BlockSpec(memory_space=_ANY)],
        out_specs=pl.BlockSpec(memory_space=pltpu.VMEM),
        scratch_shapes=[
            pltpu.VMEM((2, mh, ch), jnp.float32),
            pltpu.VMEM((2, mh, ch), jnp.bfloat16),
            pltpu.VMEM((N_DEV - 1, mh, ch), jnp.bfloat16),
            pltpu.VMEM((N_DEV - 1, mh, ch), jnp.bfloat16),
            pltpu.SemaphoreType.DMA((2, SUB)),
            pltpu.SemaphoreType.DMA((2, N_DEV - 1, SUB)),
            pltpu.SemaphoreType.DMA((2, N_DEV - 1, SUB)),
        ],
        compiler_params=_CompilerParams(
            collective_id=0, vmem_limit_bytes=60 * 1024 * 1024
        ),
    )(x)


# device time: 153896 ns/iter; 2.1460x vs baseline; 1.0072x over previous
import jax
import jax.numpy as jnp
from jax import lax
from jax.experimental import pallas as pl
from jax.experimental.pallas import tpu as pltpu

N_DEV = 4
SUB = 8

_DeviceIdType = getattr(pl, "DeviceIdType", None) or pltpu.DeviceIdType
_sem_signal = getattr(pl, "semaphore_signal", None) or pltpu.semaphore_signal
_sem_wait = getattr(pl, "semaphore_wait", None) or pltpu.semaphore_wait
_CompilerParams = getattr(pltpu, "CompilerParams", None) or getattr(
    pltpu, "TPUCompilerParams"
)
_ANY = getattr(pltpu, "ANY", None) or pl.ANY


def kernel(x):
    _, m, n = x.shape
    ch = n // N_DEV
    mh = m // 2
    qm = mh // SUB

    def body(
        x_hbm,
        out_ref,
        stage_ref,
        send0_ref,
        comm_cw,
        comm_ccw,
        local_sems,
        send_sems,
        recv_sems,
    ):
        p = lax.axis_index("i")
        left = lax.rem(p + N_DEV - 1, N_DEV)
        right = lax.rem(p + 1, N_DEV)
        comm = (comm_cw, comm_ccw)
        peer = (right, left)

        def col(c):
            return pl.ds(c * ch, ch)

        def rows(s):
            return pl.ds(s * qm, qm)

        def send_chunk(d, h):
            return lax.rem(p + 3 - h + N_DEV, N_DEV) if d == 0 else lax.rem(
                p + 1 + h, N_DEV
            )

        def recv_chunk(d, h):
            return lax.rem(p + 2 - h + N_DEV, N_DEV) if d == 0 else lax.rem(
                p + 2 + h, N_DEV
            )

        def load(d, s, c):
            return pltpu.make_async_copy(
                x_hbm.at[0, pl.ds(d * mh + s * qm, qm), col(c)],
                stage_ref.at[d, rows(s)],
                local_sems.at[d, s],
            )

        def make_rdma(d, h, s):
            src = send0_ref.at[d, rows(s)] if h == 0 else comm[d].at[h - 1, rows(s)]
            return pltpu.make_async_remote_copy(
                src_ref=src,
                dst_ref=comm[d].at[h, rows(s)],
                send_sem=send_sems.at[d, h, s],
                recv_sem=recv_sems.at[d, h, s],
                device_id=(peer[d],),
                device_id_type=_DeviceIdType.MESH,
            )

        rdmas = {}
        loads = {}

        for d in (0, 1):
            for s in range(SUB):
                loads[(d, -1, s)] = load(d, s, send_chunk(d, 0))
                loads[(d, -1, s)].start()

        barrier_sem = pltpu.get_barrier_semaphore()
        for nbr in (left, right):
            _sem_signal(
                barrier_sem,
                inc=1,
                device_id=(nbr,),
                device_id_type=_DeviceIdType.MESH,
            )
        _sem_wait(barrier_sem, 2)

        for s in range(SUB):
            for d in (0, 1):
                loads[(d, -1, s)].wait()
                send0_ref[d, rows(s)] = stage_ref[d, rows(s)].astype(jnp.bfloat16)
                rdmas[(d, 0, s)] = make_rdma(d, 0, s)
                rdmas[(d, 0, s)].start()
                loads[(d, 0, s)] = load(d, s, recv_chunk(d, 0))
                loads[(d, 0, s)].start()

        for h in range(N_DEV - 1):
            for s in range(SUB):
                for d in (0, 1):
                    rdmas[(d, h, s)].wait_recv()
                    loads[(d, h, s)].wait()
                    acc = comm[d][h, rows(s)] + stage_ref[d, rows(s)].astype(
                        jnp.bfloat16
                    )
                    if h < N_DEV - 2:
                        comm[d][h, rows(s)] = acc
                        rdmas[(d, h + 1, s)] = make_rdma(d, h + 1, s)
                        rdmas[(d, h + 1, s)].start()
                        loads[(d, h + 1, s)] = load(d, s, recv_chunk(d, h + 1))
                        loads[(d, h + 1, s)].start()
                    else:
                        out_ref[pl.ds(d * mh + s * qm, qm), :] = acc

        for key in rdmas:
            rdmas[key].wait_send()

    return pl.pallas_call(
        body,
        out_shape=jax.ShapeDtypeStruct((m, ch), jnp.bfloat16),
        in_specs=[pl.BlockSpec(memory_space=_ANY)],
        out_specs=pl.BlockSpec(memory_space=pltpu.VMEM),
        scratch_shapes=[
            pltpu.VMEM((2, mh, ch), jnp.float32),
            pltpu.VMEM((2, mh, ch), jnp.bfloat16),
            pltpu.VMEM((N_DEV - 1, mh, ch), jnp.bfloat16),
            pltpu.VMEM((N_DEV - 1, mh, ch), jnp.bfloat16),
            pltpu.SemaphoreType.DMA((2, SUB)),
            pltpu.SemaphoreType.DMA((2, N_DEV - 1, SUB)),
            pltpu.SemaphoreType.DMA((2, N_DEV - 1, SUB)),
        ],
        compiler_params=_CompilerParams(
            collective_id=0, vmem_limit_bytes=60 * 1024 * 1024
        ),
    )(x)
